# baseline (speedup 1.0000x reference)
import functools

import jax
import jax.numpy as jnp
from jax import lax
from jax.experimental import pallas as pl
from jax.experimental.pallas import tpu as pltpu
from jax.experimental.pallas import tpu_sc as plsc

_NE = 1024
_ED = 256
_B = 16
_H = 576
_BETA = 0.25

_LANES = 16
_HGRP = _H // _LANES
_CCHUNK = 64
_NCHUNK = (_ED // 2) // _CCHUNK


def _tc_body(z_ref, e2_ref, esq_ref, ones_ref,
             idx_ref, loss_ref, ppl_ref,
             counts_ref, acc_ref):
    b = pl.program_id(0)
    emb2 = e2_ref[...]
    zb = z_ref[0]

    zsq = jnp.sum(zb * zb, axis=0, keepdims=True)
    mm2 = jax.lax.dot_general(emb2, zb, (((1,), (0,)), ((), ())),
                              preferred_element_type=jnp.float32)
    d = (-zsq - esq_ref[...]) + mm2

    m = jnp.max(d, axis=0, keepdims=True)
    iota = jax.lax.broadcasted_iota(jnp.int32, (_NE, _H), 0)
    idx = jnp.max(jnp.where(d == m, iota, -1), axis=0)
    idx_ref[0, 0] = idx

    half_hot = jnp.where(iota == idx[None, :], 0.5, 0.0)
    cnt = jax.lax.dot_general(half_hot, ones_ref[...], (((1,), (0,)), ((), ())),
                              preferred_element_type=jnp.float32)
    msum = jnp.sum(m, axis=1, keepdims=True)

    @pl.when(b == 0)
    def _init():
        counts_ref[...] = cnt
        acc_ref[...] = msum

    @pl.when(b > 0)
    def _accum():
        counts_ref[...] += cnt
        acc_ref[...] += msum

    @pl.when(b == _B - 1)
    def _finalize():
        loss_ref[...] = (-(1.0 + _BETA) / (_B * _H * _ED)) * acc_ref[...]
        p = counts_ref[...] * (2.0 / (_B * _H))
        ppl_ref[...] = jnp.exp(-jnp.sum(p * jnp.log(p + 1e-10),
                                        axis=0, keepdims=True))


def _tc_select(z, emb2, esq, ones):
    return pl.pallas_call(
        _tc_body,
        grid=(_B,),
        in_specs=[
            pl.BlockSpec((1, _ED, _H), lambda b: (b, 0, 0)),
            pl.BlockSpec((_NE, _ED), lambda b: (0, 0)),
            pl.BlockSpec((_NE, 1), lambda b: (0, 0)),
            pl.BlockSpec((_H, 1), lambda b: (0, 0)),
        ],
        out_specs=[
            pl.BlockSpec((1, 1, _H), lambda b: (b, 0, 0)),
            pl.BlockSpec((1, 1), lambda b: (0, 0)),
            pl.BlockSpec((1, 1), lambda b: (0, 0)),
        ],
        out_shape=[
            jax.ShapeDtypeStruct((_B, 1, _H), jnp.int32),
            jax.ShapeDtypeStruct((1, 1), jnp.float32),
            jax.ShapeDtypeStruct((1, 1), jnp.float32),
        ],
        scratch_shapes=[
            pltpu.VMEM((_NE, 1), jnp.float32),
            pltpu.VMEM((1, 1), jnp.float32),
        ],
        compiler_params=pltpu.CompilerParams(
            dimension_semantics=("arbitrary",)),
    )(z, emb2, esq, ones)


def _sc_gather_body(embt_hbm, idx_hbm, out_hbm, idxv, ev, ov):
    b = lax.axis_index("s")
    half = lax.axis_index("c")

    pltpu.sync_copy(idx_hbm.at[pl.ds(b * _H, _H)], idxv)

    def chunk_body(chunk, carry):
        c0 = half * (_ED // 2) + chunk * _CCHUNK
        pltpu.sync_copy(embt_hbm.at[pl.ds(c0 * _NE, _CCHUNK * _NE)], ev)

        def row_body(i, carry2):
            base = i * _NE
            obase = i * _H
            for g in range(_HGRP):
                iv = idxv[pl.ds(g * _LANES, _LANES)]
                vals = plsc.load_gather(ev, [iv + base])
                ov[pl.ds(obase + g * _LANES, _LANES)] = vals
            return carry2

        lax.fori_loop(0, _CCHUNK, row_body, 0)
        pltpu.sync_copy(
            ov, out_hbm.at[pl.ds(b * (_ED * _H) + c0 * _H, _CCHUNK * _H)])
        return carry

    lax.fori_loop(0, _NCHUNK, chunk_body, 0)


@functools.partial(
    pl.kernel,
    out_type=jax.ShapeDtypeStruct((_B * _ED * _H,), jnp.float32),
    mesh=plsc.VectorSubcoreMesh(core_axis_name="c", subcore_axis_name="s"),
    scratch_types=[
        pltpu.VMEM((_H,), jnp.int32),
        pltpu.VMEM((_CCHUNK * _NE,), jnp.float32),
        pltpu.VMEM((_CCHUNK * _H,), jnp.float32),
    ],
    compiler_params=pltpu.CompilerParams(needs_layout_passes=False),
)
def _sc_gather(embt_hbm, idx_hbm, out_hbm, idxv, ev, ov):
    _sc_gather_body(embt_hbm, idx_hbm, out_hbm, idxv, ev, ov)


def kernel(z, embedding):
    emb2 = embedding + embedding
    esq = jnp.sum(embedding ** 2, axis=1).reshape(_NE, 1)
    ones = jnp.ones((_H, 1), jnp.float32)

    idx3, loss, ppl = _tc_select(z, emb2, esq, ones)

    embqt = jax.lax.dot_general(embedding, jnp.eye(_NE, dtype=jnp.float32),
                                (((0,), (0,)), ((), ())))
    embqt_flat = embqt.reshape(-1)
    zq = _sc_gather(embqt_flat, idx3.reshape(-1)).reshape(_B, _ED, _H)

    return (zq, loss[0, 0], idx3.reshape(_B, _H), ppl[0, 0])


# speedup vs baseline: 1.3336x; 1.3336x over previous
import functools

import jax
import jax.numpy as jnp
from jax import lax
from jax.experimental import pallas as pl
from jax.experimental.pallas import tpu as pltpu
from jax.experimental.pallas import tpu_sc as plsc

_NE = 1024
_ED = 256
_B = 16
_H = 576
_BETA = 0.25

_LANES = 16
_HGRP = _H // _LANES
_CCHUNK = 64
_NCHUNK = (_ED // 2) // _CCHUNK


def _tc_body(z_ref, e2_ref, esq_ref, ones_ref,
             idx_ref, loss_ref, ppl_ref,
             counts_ref, acc_ref):
    b = pl.program_id(0)
    emb2 = e2_ref[...]
    zb = z_ref[0]

    zsq = jnp.sum(zb * zb, axis=0, keepdims=True)
    mm2 = jax.lax.dot_general(emb2, zb, (((1,), (0,)), ((), ())),
                              preferred_element_type=jnp.float32)
    d = (-zsq - esq_ref[...]) + mm2

    m = jnp.max(d, axis=0, keepdims=True)
    iota = jax.lax.broadcasted_iota(jnp.int32, (_NE, _H), 0)
    idx = jnp.max(jnp.where(d == m, iota, -1), axis=0)
    idx_ref[0, 0] = idx

    half_hot = jnp.where(iota == idx[None, :], 0.5, 0.0)
    cnt = jax.lax.dot_general(half_hot, ones_ref[...], (((1,), (0,)), ((), ())),
                              preferred_element_type=jnp.float32)
    msum = jnp.sum(m, axis=1, keepdims=True)

    @pl.when(b == 0)
    def _init():
        counts_ref[...] = cnt
        acc_ref[...] = msum

    @pl.when(b > 0)
    def _accum():
        counts_ref[...] += cnt
        acc_ref[...] += msum

    @pl.when(b == _B - 1)
    def _finalize():
        loss_ref[...] = (-(1.0 + _BETA) / (_B * _H * _ED)) * acc_ref[...]
        p = counts_ref[...] * (2.0 / (_B * _H))
        ppl_ref[...] = jnp.exp(-jnp.sum(p * jnp.log(p + 1e-10),
                                        axis=0, keepdims=True))


def _tc_select(z, emb2, esq, ones):
    return pl.pallas_call(
        _tc_body,
        grid=(_B,),
        in_specs=[
            pl.BlockSpec((1, _ED, _H), lambda b: (b, 0, 0)),
            pl.BlockSpec((_NE, _ED), lambda b: (0, 0)),
            pl.BlockSpec((_NE, 1), lambda b: (0, 0)),
            pl.BlockSpec((_H, 1), lambda b: (0, 0)),
        ],
        out_specs=[
            pl.BlockSpec((1, 1, _H), lambda b: (b, 0, 0)),
            pl.BlockSpec((1, 1), lambda b: (0, 0)),
            pl.BlockSpec((1, 1), lambda b: (0, 0)),
        ],
        out_shape=[
            jax.ShapeDtypeStruct((_B, 1, _H), jnp.int32),
            jax.ShapeDtypeStruct((1, 1), jnp.float32),
            jax.ShapeDtypeStruct((1, 1), jnp.float32),
        ],
        scratch_shapes=[
            pltpu.VMEM((_NE, 1), jnp.float32),
            pltpu.VMEM((1, 1), jnp.float32),
        ],
        compiler_params=pltpu.CompilerParams(
            dimension_semantics=("arbitrary",)),
    )(z, emb2, esq, ones)


def _sc_gather_body(embt_hbm, idx_hbm, out_hbm, idxv, ev, ov):
    b = lax.axis_index("s")
    half = lax.axis_index("c")

    pltpu.sync_copy(idx_hbm.at[pl.ds(b * _H, _H)], idxv)

    def chunk_body(chunk, carry):
        c0 = half * (_ED // 2) + chunk * _CCHUNK
        pltpu.sync_copy(embt_hbm.at[pl.ds(c0 * _NE, _CCHUNK * _NE)], ev)

        @plsc.parallel_loop(0, _CCHUNK, unroll=2)
        def row_body(i):
            base = i * _NE
            obase = i * _H
            for g in range(_HGRP):
                iv = idxv[pl.ds(g * _LANES, _LANES)]
                vals = plsc.load_gather(ev, [iv + base])
                ov[pl.ds(obase + g * _LANES, _LANES)] = vals
        pltpu.sync_copy(
            ov, out_hbm.at[pl.ds(b * (_ED * _H) + c0 * _H, _CCHUNK * _H)])
        return carry

    lax.fori_loop(0, _NCHUNK, chunk_body, 0)


@functools.partial(
    pl.kernel,
    out_type=jax.ShapeDtypeStruct((_B * _ED * _H,), jnp.float32),
    mesh=plsc.VectorSubcoreMesh(core_axis_name="c", subcore_axis_name="s"),
    scratch_types=[
        pltpu.VMEM((_H,), jnp.int32),
        pltpu.VMEM((_CCHUNK * _NE,), jnp.float32),
        pltpu.VMEM((_CCHUNK * _H,), jnp.float32),
    ],
    compiler_params=pltpu.CompilerParams(needs_layout_passes=False),
)
def _sc_gather(embt_hbm, idx_hbm, out_hbm, idxv, ev, ov):
    _sc_gather_body(embt_hbm, idx_hbm, out_hbm, idxv, ev, ov)


def kernel(z, embedding):
    emb2 = embedding + embedding
    esq = jnp.sum(embedding ** 2, axis=1).reshape(_NE, 1)
    ones = jnp.ones((_H, 1), jnp.float32)

    idx3, loss, ppl = _tc_select(z, emb2, esq, ones)

    embqt = jax.lax.dot_general(embedding, jnp.eye(_NE, dtype=jnp.float32),
                                (((0,), (0,)), ((), ())))
    embqt_flat = embqt.reshape(-1)
    zq = _sc_gather(embqt_flat, idx3.reshape(-1)).reshape(_B, _ED, _H)

    return (zq, loss[0, 0], idx3.reshape(_B, _H), ppl[0, 0])
